# baseline (speedup 1.0000x reference)
import functools

import jax
import jax.numpy as jnp
from jax import lax
from jax.experimental import pallas as pl
from jax.experimental.pallas import tpu as pltpu
from jax.experimental.pallas import tpu_sc as plsc

SR = 44100
MIN_DUR = 24
MAX_DUR = 600
N_TIME = 2048
T_BINS = 128
OUT_W = 64
TOT_MIN = float(MIN_DUR * SR)
TOT_MAX = float(MAX_DUR * SR)

TB = 512


def _start_emb_sc(aidx, gidx, artist_w, bow_aug, n, k, w):
    info = plsc.get_sparse_core_info()
    nw = info.num_cores * info.num_subcores
    b = n // nw
    mesh = plsc.VectorSubcoreMesh(core_axis_name="c", subcore_axis_name="s")

    @functools.partial(
        pl.kernel,
        mesh=mesh,
        out_type=jax.ShapeDtypeStruct((n, w), jnp.float32),
        scratch_types=[
            pltpu.VMEM((b,), jnp.int32),
            pltpu.VMEM((b * k,), jnp.int32),
            pltpu.VMEM((b, w), jnp.float32),
            pltpu.VMEM((b * k, w), jnp.float32),
            pltpu.VMEM((b, w), jnp.float32),
            pltpu.SemaphoreType.DMA,
        ],
        compiler_params=pltpu.CompilerParams(use_tc_tiling_on_sc=False),
    )
    def sc_kernel(aidx_hbm, gidx_hbm, aw_hbm, bw_hbm, out_hbm,
                  aidx_v, gidx_v, arows, grows, acc, sem):
        wid = lax.axis_index("s") * info.num_cores + lax.axis_index("c")
        pltpu.sync_copy(aidx_hbm.at[pl.ds(wid * b, b)], aidx_v)
        pltpu.sync_copy(gidx_hbm.at[pl.ds(wid * b * k, b * k)], gidx_v)
        pltpu.async_copy(aw_hbm.at[aidx_v], arows, sem).wait()
        pltpu.async_copy(bw_hbm.at[gidx_v], grows, sem).wait()
        for i in range(b):
            for wv in range(w // 16):
                s = arows[i, pl.ds(wv * 16, 16)]
                for j in range(k):
                    s = s + grows[i * k + j, pl.ds(wv * 16, 16)]
                acc[i, pl.ds(wv * 16, 16)] = s
        pltpu.sync_copy(acc, out_hbm.at[pl.ds(wid * b, b)])

    return sc_kernel(aidx, gidx, artist_w, bow_aug)


def _pos_body(y_ref, dw_ref, dwt_ref, o_ref):
    n = pl.program_id(0)
    tot = y_ref[n, 0].astype(jnp.float32)
    off_i = y_ref[n, 1]
    len_i = y_ref[n, 2]
    start = off_i.astype(jnp.float32)
    end = (off_i + len_i).astype(jnp.float32)

    lanes = lax.broadcasted_iota(jnp.int32, (1, T_BINS), 1)
    t0 = pl.program_id(1) * TB
    interp = ((lax.broadcasted_iota(jnp.int32, (TB, 1), 0) + t0)
              .astype(jnp.float32) / float(N_TIME))
    pos_a = start + (end - start) * interp
    xs_a = T_BINS * (pos_a / TOT_MAX)
    ps_r = start / tot
    pe_r = jnp.clip(end / tot, 0.0, 1.0)
    pos_r = ps_r + (pe_r - ps_r) * interp
    xs_r = T_BINS * pos_r

    bins_a = jnp.minimum(jnp.floor(xs_a).astype(jnp.int32), T_BINS - 1)
    bins_r = jnp.minimum(jnp.floor(xs_r).astype(jnp.int32), T_BINS - 1)
    oh = jnp.concatenate([(bins_a == lanes).astype(jnp.bfloat16),
                          (bins_r == lanes).astype(jnp.bfloat16)], axis=1)
    emb = jnp.dot(oh, dw_ref[...], preferred_element_type=jnp.float32)

    xt = T_BINS * ((tot - TOT_MIN) / (TOT_MAX - TOT_MIN))
    bin_t = jnp.minimum(jnp.floor(xt).astype(jnp.int32), T_BINS - 1)
    oh_t = (bin_t == lanes).astype(jnp.bfloat16)
    embt = jnp.dot(oh_t, dwt_ref[...], preferred_element_type=jnp.float32)

    o_ref[0] = emb + embt


def _pos_emb_tc(y, total_w, abs_w, rel_w, n, w, interpret=False):
    dw = jnp.concatenate([abs_w, rel_w], axis=0).astype(jnp.bfloat16)
    dwt = total_w.astype(jnp.bfloat16)
    grid = (n, N_TIME // TB)
    return pl.pallas_call(
        _pos_body,
        grid=grid,
        in_specs=[
            pl.BlockSpec(memory_space=pltpu.SMEM),
            pl.BlockSpec((2 * T_BINS, w), lambda i, j: (0, 0)),
            pl.BlockSpec((T_BINS, w), lambda i, j: (0, 0)),
        ],
        out_specs=pl.BlockSpec((1, TB, w), lambda i, j: (i, j, 0)),
        out_shape=jax.ShapeDtypeStruct((n, N_TIME, w), jnp.float32),
        compiler_params=pltpu.CompilerParams(
            dimension_semantics=("parallel", "parallel"),
            vmem_limit_bytes=100 * 1024 * 1024),
        interpret=interpret,
    )(y, dw, dwt)


def kernel(y, bow_genre_w, artist_w, total_w, abs_w, rel_w):
    n = y.shape[0]
    k = y.shape[1] - 4
    g = bow_genre_w.shape[0]
    w = bow_genre_w.shape[1]

    artist_idx = y[:, 3].astype(jnp.int32)
    genre = y[:, 4:]
    gidx = jnp.where(genre < 0, g, genre).astype(jnp.int32).reshape(-1)
    bow_aug = jnp.concatenate(
        [bow_genre_w, jnp.zeros((1, w), jnp.float32)], axis=0)

    start2d = _start_emb_sc(artist_idx, gidx, artist_w, bow_aug, n, k, w)
    start_emb = start2d.reshape(n, 1, w)
    pos_emb = _pos_emb_tc(y, total_w, abs_w, rel_w, n, w)
    return (start_emb, pos_emb)


# speedup vs baseline: 1.7672x; 1.7672x over previous
import functools

import jax
import jax.numpy as jnp
from jax import lax
from jax.experimental import pallas as pl
from jax.experimental.pallas import tpu as pltpu
from jax.experimental.pallas import tpu_sc as plsc

SR = 44100
MIN_DUR = 24
MAX_DUR = 600
N_TIME = 2048
T_BINS = 128
OUT_W = 64
TOT_MIN = float(MIN_DUR * SR)
TOT_MAX = float(MAX_DUR * SR)

TB = 2048
BN = 4


def _start_emb_sc(aidx, gidx, artist_w, bow_aug, n, k, w):
    info = plsc.get_sparse_core_info()
    nw = info.num_cores * info.num_subcores
    b = n // nw
    mesh = plsc.VectorSubcoreMesh(core_axis_name="c", subcore_axis_name="s")

    @functools.partial(
        pl.kernel,
        mesh=mesh,
        out_type=jax.ShapeDtypeStruct((n, w), jnp.float32),
        scratch_types=[
            pltpu.VMEM((b,), jnp.int32),
            pltpu.VMEM((b * k,), jnp.int32),
            pltpu.VMEM((b, w), jnp.float32),
            pltpu.VMEM((b * k, w), jnp.float32),
            pltpu.VMEM((b, w), jnp.float32),
            pltpu.SemaphoreType.DMA,
        ],
        compiler_params=pltpu.CompilerParams(use_tc_tiling_on_sc=False),
    )
    def sc_kernel(aidx_hbm, gidx_hbm, aw_hbm, bw_hbm, out_hbm,
                  aidx_v, gidx_v, arows, grows, acc, sem):
        wid = lax.axis_index("s") * info.num_cores + lax.axis_index("c")
        pltpu.sync_copy(aidx_hbm.at[pl.ds(wid * b, b)], aidx_v)
        pltpu.sync_copy(gidx_hbm.at[pl.ds(wid * b * k, b * k)], gidx_v)
        pltpu.async_copy(aw_hbm.at[aidx_v], arows, sem).wait()
        pltpu.async_copy(bw_hbm.at[gidx_v], grows, sem).wait()
        for i in range(b):
            for wv in range(w // 16):
                s = arows[i, pl.ds(wv * 16, 16)]
                for j in range(k):
                    s = s + grows[i * k + j, pl.ds(wv * 16, 16)]
                acc[i, pl.ds(wv * 16, 16)] = s
        pltpu.sync_copy(acc, out_hbm.at[pl.ds(wid * b, b)])

    return sc_kernel(aidx, gidx, artist_w, bow_aug)


def _pos_body(y_ref, dw_ref, dwt_ref, o_ref):
    lanes = lax.broadcasted_iota(jnp.int32, (1, T_BINS), 1)
    interp = (lax.broadcasted_iota(jnp.int32, (TB, 1), 0).astype(jnp.float32)
              / float(N_TIME))
    for ns in range(BN):
        n = pl.program_id(0) * BN + ns
        tot = y_ref[n, 0].astype(jnp.float32)
        off_i = y_ref[n, 1]
        len_i = y_ref[n, 2]
        start = off_i.astype(jnp.float32)
        end = (off_i + len_i).astype(jnp.float32)

        pos_a = start + (end - start) * interp
        xs_a = T_BINS * (pos_a / TOT_MAX)
        ps_r = start / tot
        pe_r = jnp.clip(end / tot, 0.0, 1.0)
        pos_r = ps_r + (pe_r - ps_r) * interp
        xs_r = T_BINS * pos_r

        bins_a = jnp.minimum(jnp.floor(xs_a).astype(jnp.int32), T_BINS - 1)
        bins_r = jnp.minimum(jnp.floor(xs_r).astype(jnp.int32), T_BINS - 1)
        oh = jnp.concatenate([(bins_a == lanes).astype(jnp.bfloat16),
                              (bins_r == lanes).astype(jnp.bfloat16)], axis=1)
        emb = jnp.dot(oh, dw_ref[...], preferred_element_type=jnp.float32)

        xt = T_BINS * ((tot - TOT_MIN) / (TOT_MAX - TOT_MIN))
        bin_t = jnp.minimum(jnp.floor(xt).astype(jnp.int32), T_BINS - 1)
        oh_t = (bin_t == lanes).astype(jnp.bfloat16)
        embt = jnp.dot(oh_t, dwt_ref[...], preferred_element_type=jnp.float32)

        o_ref[ns] = emb + embt


def _pos_emb_tc(y, total_w, abs_w, rel_w, n, w, interpret=False):
    dw = jnp.concatenate([abs_w, rel_w], axis=0).astype(jnp.bfloat16)
    dwt = total_w.astype(jnp.bfloat16)
    grid = (n // BN,)
    return pl.pallas_call(
        _pos_body,
        grid=grid,
        in_specs=[
            pl.BlockSpec(memory_space=pltpu.SMEM),
            pl.BlockSpec((2 * T_BINS, w), lambda i: (0, 0)),
            pl.BlockSpec((T_BINS, w), lambda i: (0, 0)),
        ],
        out_specs=pl.BlockSpec((BN, TB, w), lambda i: (i, 0, 0)),
        out_shape=jax.ShapeDtypeStruct((n, N_TIME, w), jnp.float32),
        compiler_params=pltpu.CompilerParams(
            dimension_semantics=("parallel",),
            vmem_limit_bytes=100 * 1024 * 1024),
        interpret=interpret,
    )(y, dw, dwt)


def kernel(y, bow_genre_w, artist_w, total_w, abs_w, rel_w):
    n = y.shape[0]
    k = y.shape[1] - 4
    g = bow_genre_w.shape[0]
    w = bow_genre_w.shape[1]

    artist_idx = y[:, 3].astype(jnp.int32)
    genre = y[:, 4:]
    gidx = jnp.where(genre < 0, g, genre).astype(jnp.int32).reshape(-1)
    bow_aug = jnp.concatenate(
        [bow_genre_w, jnp.zeros((1, w), jnp.float32)], axis=0)

    start2d = _start_emb_sc(artist_idx, gidx, artist_w, bow_aug, n, k, w)
    start_emb = start2d.reshape(n, 1, w)
    pos_emb = _pos_emb_tc(y, total_w, abs_w, rel_w, n, w)
    return (start_emb, pos_emb)


# speedup vs baseline: 1.7979x; 1.0174x over previous
import functools

import jax
import jax.numpy as jnp
from jax import lax
from jax.experimental import pallas as pl
from jax.experimental.pallas import tpu as pltpu
from jax.experimental.pallas import tpu_sc as plsc

SR = 44100
MIN_DUR = 24
MAX_DUR = 600
N_TIME = 2048
T_BINS = 128
OUT_W = 64
TOT_MIN = float(MIN_DUR * SR)
TOT_MAX = float(MAX_DUR * SR)

TB = 2048
BN = 8


def _start_emb_sc(aidx, gidx, artist_w, bow_aug, n, k, w):
    info = plsc.get_sparse_core_info()
    nw = info.num_cores * info.num_subcores
    b = n // nw
    mesh = plsc.VectorSubcoreMesh(core_axis_name="c", subcore_axis_name="s")

    @functools.partial(
        pl.kernel,
        mesh=mesh,
        out_type=jax.ShapeDtypeStruct((n, w), jnp.float32),
        scratch_types=[
            pltpu.VMEM((b,), jnp.int32),
            pltpu.VMEM((b * k,), jnp.int32),
            pltpu.VMEM((b, w), jnp.float32),
            pltpu.VMEM((b * k, w), jnp.float32),
            pltpu.VMEM((b, w), jnp.float32),
            pltpu.SemaphoreType.DMA,
        ],
        compiler_params=pltpu.CompilerParams(use_tc_tiling_on_sc=False),
    )
    def sc_kernel(aidx_hbm, gidx_hbm, aw_hbm, bw_hbm, out_hbm,
                  aidx_v, gidx_v, arows, grows, acc, sem):
        wid = lax.axis_index("s") * info.num_cores + lax.axis_index("c")
        pltpu.sync_copy(aidx_hbm.at[pl.ds(wid * b, b)], aidx_v)
        pltpu.sync_copy(gidx_hbm.at[pl.ds(wid * b * k, b * k)], gidx_v)
        pltpu.async_copy(aw_hbm.at[aidx_v], arows, sem).wait()
        pltpu.async_copy(bw_hbm.at[gidx_v], grows, sem).wait()
        for i in range(b):
            for wv in range(w // 16):
                s = arows[i, pl.ds(wv * 16, 16)]
                for j in range(k):
                    s = s + grows[i * k + j, pl.ds(wv * 16, 16)]
                acc[i, pl.ds(wv * 16, 16)] = s
        pltpu.sync_copy(acc, out_hbm.at[pl.ds(wid * b, b)])

    return sc_kernel(aidx, gidx, artist_w, bow_aug)


def _pos_body(y_ref, dw_ref, dwt_ref, o_ref):
    lanes = lax.broadcasted_iota(jnp.int32, (1, T_BINS), 1)
    interp = (lax.broadcasted_iota(jnp.int32, (TB, 1), 0).astype(jnp.float32)
              / float(N_TIME))
    for ns in range(BN):
        n = pl.program_id(0) * BN + ns
        tot = y_ref[n, 0].astype(jnp.float32)
        off_i = y_ref[n, 1]
        len_i = y_ref[n, 2]
        start = off_i.astype(jnp.float32)
        end = (off_i + len_i).astype(jnp.float32)

        pos_a = start + (end - start) * interp
        xs_a = T_BINS * (pos_a / TOT_MAX)
        ps_r = start / tot
        pe_r = jnp.clip(end / tot, 0.0, 1.0)
        pos_r = ps_r + (pe_r - ps_r) * interp
        xs_r = T_BINS * pos_r

        bins_a = jnp.minimum(jnp.floor(xs_a).astype(jnp.int32), T_BINS - 1)
        bins_r = jnp.minimum(jnp.floor(xs_r).astype(jnp.int32), T_BINS - 1)
        oh = jnp.concatenate([(bins_a == lanes).astype(jnp.bfloat16),
                              (bins_r == lanes).astype(jnp.bfloat16)], axis=1)
        emb = jnp.dot(oh, dw_ref[...], preferred_element_type=jnp.float32)

        xt = T_BINS * ((tot - TOT_MIN) / (TOT_MAX - TOT_MIN))
        bin_t = jnp.minimum(jnp.floor(xt).astype(jnp.int32), T_BINS - 1)
        oh_t = (bin_t == lanes).astype(jnp.bfloat16)
        embt = jnp.dot(oh_t, dwt_ref[...], preferred_element_type=jnp.float32)

        o_ref[ns] = emb + embt


def _pos_emb_tc(y, total_w, abs_w, rel_w, n, w, interpret=False):
    dw = jnp.concatenate([abs_w, rel_w], axis=0).astype(jnp.bfloat16)
    dwt = total_w.astype(jnp.bfloat16)
    grid = (n // BN,)
    return pl.pallas_call(
        _pos_body,
        grid=grid,
        in_specs=[
            pl.BlockSpec(memory_space=pltpu.SMEM),
            pl.BlockSpec((2 * T_BINS, w), lambda i: (0, 0)),
            pl.BlockSpec((T_BINS, w), lambda i: (0, 0)),
        ],
        out_specs=pl.BlockSpec((BN, TB, w), lambda i: (i, 0, 0)),
        out_shape=jax.ShapeDtypeStruct((n, N_TIME, w), jnp.float32),
        compiler_params=pltpu.CompilerParams(
            dimension_semantics=("parallel",),
            vmem_limit_bytes=100 * 1024 * 1024),
        interpret=interpret,
    )(y, dw, dwt)


def kernel(y, bow_genre_w, artist_w, total_w, abs_w, rel_w):
    n = y.shape[0]
    k = y.shape[1] - 4
    g = bow_genre_w.shape[0]
    w = bow_genre_w.shape[1]

    artist_idx = y[:, 3].astype(jnp.int32)
    genre = y[:, 4:]
    gidx = jnp.where(genre < 0, g, genre).astype(jnp.int32).reshape(-1)
    bow_aug = jnp.concatenate(
        [bow_genre_w, jnp.zeros((1, w), jnp.float32)], axis=0)

    start2d = _start_emb_sc(artist_idx, gidx, artist_w, bow_aug, n, k, w)
    start_emb = start2d.reshape(n, 1, w)
    pos_emb = _pos_emb_tc(y, total_w, abs_w, rel_w, n, w)
    return (start_emb, pos_emb)


# speedup vs baseline: 1.8053x; 1.0042x over previous
import functools

import jax
import jax.numpy as jnp
from jax import lax
from jax.experimental import pallas as pl
from jax.experimental.pallas import tpu as pltpu
from jax.experimental.pallas import tpu_sc as plsc

SR = 44100
MIN_DUR = 24
MAX_DUR = 600
N_TIME = 2048
T_BINS = 128
OUT_W = 64
TOT_MIN = float(MIN_DUR * SR)
TOT_MAX = float(MAX_DUR * SR)

TB = 2048
BN = 16


def _start_emb_sc(aidx, gidx, artist_w, bow_aug, n, k, w):
    info = plsc.get_sparse_core_info()
    nw = info.num_cores * info.num_subcores
    b = n // nw
    mesh = plsc.VectorSubcoreMesh(core_axis_name="c", subcore_axis_name="s")

    @functools.partial(
        pl.kernel,
        mesh=mesh,
        out_type=jax.ShapeDtypeStruct((n, w), jnp.float32),
        scratch_types=[
            pltpu.VMEM((b,), jnp.int32),
            pltpu.VMEM((b * k,), jnp.int32),
            pltpu.VMEM((b, w), jnp.float32),
            pltpu.VMEM((b * k, w), jnp.float32),
            pltpu.VMEM((b, w), jnp.float32),
            pltpu.SemaphoreType.DMA,
        ],
        compiler_params=pltpu.CompilerParams(use_tc_tiling_on_sc=False),
    )
    def sc_kernel(aidx_hbm, gidx_hbm, aw_hbm, bw_hbm, out_hbm,
                  aidx_v, gidx_v, arows, grows, acc, sem):
        wid = lax.axis_index("s") * info.num_cores + lax.axis_index("c")
        pltpu.sync_copy(aidx_hbm.at[pl.ds(wid * b, b)], aidx_v)
        pltpu.sync_copy(gidx_hbm.at[pl.ds(wid * b * k, b * k)], gidx_v)
        pltpu.async_copy(aw_hbm.at[aidx_v], arows, sem).wait()
        pltpu.async_copy(bw_hbm.at[gidx_v], grows, sem).wait()
        for i in range(b):
            for wv in range(w // 16):
                s = arows[i, pl.ds(wv * 16, 16)]
                for j in range(k):
                    s = s + grows[i * k + j, pl.ds(wv * 16, 16)]
                acc[i, pl.ds(wv * 16, 16)] = s
        pltpu.sync_copy(acc, out_hbm.at[pl.ds(wid * b, b)])

    return sc_kernel(aidx, gidx, artist_w, bow_aug)


def _pos_body(y_ref, dw_ref, dwt_ref, o_ref):
    lanes = lax.broadcasted_iota(jnp.int32, (1, T_BINS), 1)
    interp = (lax.broadcasted_iota(jnp.int32, (TB, 1), 0).astype(jnp.float32)
              / float(N_TIME))
    for ns in range(BN):
        n = pl.program_id(0) * BN + ns
        tot = y_ref[n, 0].astype(jnp.float32)
        off_i = y_ref[n, 1]
        len_i = y_ref[n, 2]
        start = off_i.astype(jnp.float32)
        end = (off_i + len_i).astype(jnp.float32)

        pos_a = start + (end - start) * interp
        xs_a = T_BINS * (pos_a / TOT_MAX)
        ps_r = start / tot
        pe_r = jnp.clip(end / tot, 0.0, 1.0)
        pos_r = ps_r + (pe_r - ps_r) * interp
        xs_r = T_BINS * pos_r

        bins_a = jnp.minimum(jnp.floor(xs_a).astype(jnp.int32), T_BINS - 1)
        bins_r = jnp.minimum(jnp.floor(xs_r).astype(jnp.int32), T_BINS - 1)
        oh = jnp.concatenate([(bins_a == lanes).astype(jnp.bfloat16),
                              (bins_r == lanes).astype(jnp.bfloat16)], axis=1)
        emb = jnp.dot(oh, dw_ref[...], preferred_element_type=jnp.float32)

        xt = T_BINS * ((tot - TOT_MIN) / (TOT_MAX - TOT_MIN))
        bin_t = jnp.minimum(jnp.floor(xt).astype(jnp.int32), T_BINS - 1)
        oh_t = (bin_t == lanes).astype(jnp.bfloat16)
        embt = jnp.dot(oh_t, dwt_ref[...], preferred_element_type=jnp.float32)

        o_ref[ns] = emb + embt


def _pos_emb_tc(y, total_w, abs_w, rel_w, n, w, interpret=False):
    dw = jnp.concatenate([abs_w, rel_w], axis=0).astype(jnp.bfloat16)
    dwt = total_w.astype(jnp.bfloat16)
    grid = (n // BN,)
    return pl.pallas_call(
        _pos_body,
        grid=grid,
        in_specs=[
            pl.BlockSpec(memory_space=pltpu.SMEM),
            pl.BlockSpec((2 * T_BINS, w), lambda i: (0, 0)),
            pl.BlockSpec((T_BINS, w), lambda i: (0, 0)),
        ],
        out_specs=pl.BlockSpec((BN, TB, w), lambda i: (i, 0, 0)),
        out_shape=jax.ShapeDtypeStruct((n, N_TIME, w), jnp.float32),
        compiler_params=pltpu.CompilerParams(
            dimension_semantics=("parallel",),
            vmem_limit_bytes=100 * 1024 * 1024),
        interpret=interpret,
    )(y, dw, dwt)


def kernel(y, bow_genre_w, artist_w, total_w, abs_w, rel_w):
    n = y.shape[0]
    k = y.shape[1] - 4
    g = bow_genre_w.shape[0]
    w = bow_genre_w.shape[1]

    artist_idx = y[:, 3].astype(jnp.int32)
    genre = y[:, 4:]
    gidx = jnp.where(genre < 0, g, genre).astype(jnp.int32).reshape(-1)
    bow_aug = jnp.concatenate(
        [bow_genre_w, jnp.zeros((1, w), jnp.float32)], axis=0)

    start2d = _start_emb_sc(artist_idx, gidx, artist_w, bow_aug, n, k, w)
    start_emb = start2d.reshape(n, 1, w)
    pos_emb = _pos_emb_tc(y, total_w, abs_w, rel_w, n, w)
    return (start_emb, pos_emb)
